# Initial kernel scaffold; baseline (speedup 1.0000x reference)
import jax
import jax.numpy as jnp
from jax.experimental import pallas as pl


def kernel(TX, X, MX, MY, params):
    raise NotImplementedError("write your pallas kernel here")


# speedup vs baseline: 18.3600x; 18.3600x over previous
import jax
import jax.numpy as jnp
from jax.experimental import pallas as pl

_NEG = -100000000.0
_NHEADS = 2


def kernel(TX, X, MX, MY, params):
    f32 = jnp.float32
    Bn, Tn, Cn = X.shape
    L = params["chan_init"]["w"].shape[1]
    E = Tn * Cn
    dk = L // _NHEADS
    scale = 1.0 / (dk ** 0.5)

    mask = (MX + MY).astype(f32)

    def bcast(a):
        return jnp.broadcast_to(a.reshape(Bn, E, 1), (Bn, E, L))

    pack = jnp.concatenate(
        [bcast(X), bcast(MY), bcast(mask), jnp.zeros((Bn, E, 128 - 3 * L), f32)],
        axis=2)

    txb = jnp.broadcast_to(TX[:, :, None], (Bn, Tn, L))
    tnzb = jnp.broadcast_to(
        (jnp.sum(mask, axis=2, keepdims=True) > 0).astype(f32), (Bn, Tn, L))
    cnzb = jnp.broadcast_to(
        (jnp.sum(mask, axis=1)[:, :, None] > 0).astype(f32), (Bn, Cn, L))

    tw = params["time_init"]["w"]
    tb = params["time_init"]["b"][None, :]
    cw = params["chan_init"]["w"]
    cb = params["chan_init"]["b"][None, :]
    ew0 = params["edge_init"]["w"][0:1]
    ew1 = params["edge_init"]["w"][1:2]
    eb = params["edge_init"]["b"][None, :]

    def fuse_layer(lp):
        kw, kb = lp["attn"]["k"]["w"], lp["attn"]["k"]["b"]
        vw, vb = lp["attn"]["v"]["w"], lp["attn"]["v"]["b"]
        enw, enb = lp["edge_nn"]["w"], lp["edge_nn"]["b"]
        z = jnp.zeros((L, L), f32)
        wc = jnp.concatenate([
            jnp.concatenate([kw[:L], vw[:L], z, z, enw[L:2 * L]], axis=1),
            jnp.concatenate([z, z, kw[:L], vw[:L], enw[2 * L:3 * L]], axis=1),
            jnp.concatenate([kw[L:], vw[L:], kw[L:], vw[L:], enw[:L]], axis=1),
        ], axis=0)
        bc = jnp.concatenate([kb, vb, kb, vb, enb])[None, :]
        qw, qb = lp["attn"]["q"]["w"], lp["attn"]["q"]["b"][None, :]
        ow, ob = lp["attn"]["o"]["w"], lp["attn"]["o"]["b"][None, :]
        return wc, bc, qw, qb, ow, ob

    layer_ws = [w for lp in params["layers"] for w in fuse_layer(lp)]
    n_layers = len(params["layers"])

    def body(pack_ref, txb_ref, tnzb_ref, cnzb_ref,
             tw_ref, tb_ref, cw_ref, cb_ref, ew0_ref, ew1_ref, eb_ref,
             *rest):
        lw_refs = rest[:-1]
        out_ref = rest[-1]

        xb = pack_ref[0, :, 0:L]
        myb = pack_ref[0, :, L:2 * L]
        maskb = pack_ref[0, :, 2 * L:3 * L]
        mask3 = pack_ref[0, :, 2 * L:2 * L + _NHEADS].reshape(Tn, Cn, _NHEADS)
        txk = txb_ref[0]
        tnz = tnzb_ref[0]
        cnz = cnzb_ref[0]

        d_i = jax.lax.broadcasted_iota(jnp.int32, (L, _NHEADS), 0)
        h_i = jax.lax.broadcasted_iota(jnp.int32, (L, _NHEADS), 1)
        S = (d_i // dk == h_i).astype(f32)
        d_j = jax.lax.broadcasted_iota(jnp.int32, (_NHEADS, L), 1)
        h_j = jax.lax.broadcasted_iota(jnp.int32, (_NHEADS, L), 0)
        ST = (d_j // dk == h_j).astype(f32)

        T_f = jnp.sin(txk * tw_ref[...] + tb_ref[...])
        C_f = jnp.maximum(cw_ref[...] + cb_ref[...], 0.0)
        U = jnp.maximum(xb * ew0_ref[...] + myb * ew1_ref[...] + eb_ref[...],
                        0.0) * maskb

        for li in range(n_layers):
            wc, bc, qw, qb, ow, ob = (r[...] for r in lw_refs[6 * li:6 * li + 6])

            tfb = jnp.broadcast_to(T_f[:, None, :], (Tn, Cn, L)).reshape(E, L)
            cfb = jnp.broadcast_to(C_f[None, :, :], (Tn, Cn, L)).reshape(E, L)
            cin = jnp.concatenate([tfb, cfb, U], axis=1)
            P = jnp.dot(cin, wc, preferred_element_type=f32) + bc
            kC = jnp.maximum(P[:, 0:L], 0.0)
            vC = jnp.maximum(P[:, L:2 * L], 0.0)
            kT = jnp.maximum(P[:, 2 * L:3 * L], 0.0)
            vT = jnp.maximum(P[:, 3 * L:4 * L], 0.0)
            epre = P[:, 4 * L:5 * L]

            qC = jnp.maximum(jnp.dot(C_f, qw, preferred_element_type=f32) + qb,
                             0.0)
            qT = jnp.maximum(jnp.dot(T_f, qw, preferred_element_type=f32) + qb,
                             0.0)
            qTb = jnp.broadcast_to(qT[:, None, :], (Tn, Cn, L)).reshape(E, L)
            qCb = jnp.broadcast_to(qC[None, :, :], (Tn, Cn, L)).reshape(E, L)

            sT = (jnp.dot(qTb * kT, S, preferred_element_type=f32)
                  .reshape(Tn, Cn, _NHEADS)) * scale
            sT = jnp.where(mask3 > 0.0, sT, _NEG)
            eT = jnp.exp(sT - jnp.max(sT, axis=1, keepdims=True))
            awT = eT / jnp.sum(eT, axis=1, keepdims=True)
            awTb = jnp.dot(awT.reshape(E, _NHEADS), ST,
                           preferred_element_type=f32)
            avT = jnp.sum((awTb * vT).reshape(Tn, Cn, L), axis=1)
            T_new = (jnp.dot(avT, ow, preferred_element_type=f32) + ob) * tnz

            sC = (jnp.dot(qCb * kC, S, preferred_element_type=f32)
                  .reshape(Tn, Cn, _NHEADS)) * scale
            sC = jnp.where(mask3 > 0.0, sC, _NEG)
            eC = jnp.exp(sC - jnp.max(sC, axis=0, keepdims=True))
            awC = eC / jnp.sum(eC, axis=0, keepdims=True)
            awCb = jnp.dot(awC.reshape(E, _NHEADS), ST,
                           preferred_element_type=f32)
            avC = jnp.sum((awCb * vC).reshape(Tn, Cn, L), axis=0)
            C_new = (jnp.dot(avC, ow, preferred_element_type=f32) + ob) * cnz

            U = jnp.maximum(U + epre, 0.0) * maskb
            T_f = T_new
            C_f = C_new

        out_ref[0] = U

    data_specs = [
        pl.BlockSpec((1, E, 128), lambda b: (b, 0, 0)),
        pl.BlockSpec((1, Tn, L), lambda b: (b, 0, 0)),
        pl.BlockSpec((1, Tn, L), lambda b: (b, 0, 0)),
        pl.BlockSpec((1, Cn, L), lambda b: (b, 0, 0)),
    ]
    w_arrays = [tw, tb, cw, cb, ew0, ew1, eb] + layer_ws
    w_specs = [pl.BlockSpec(a.shape, lambda b: (0, 0)) for a in w_arrays]

    out = pl.pallas_call(
        body,
        grid=(Bn,),
        in_specs=data_specs + w_specs,
        out_specs=pl.BlockSpec((1, E, L), lambda b: (b, 0, 0)),
        out_shape=jax.ShapeDtypeStruct((Bn, E, L), f32),
    )(pack, txb, tnzb, cnzb, *w_arrays)
    return out.reshape(Bn, Tn, Cn, L)


# speedup vs baseline: 25.6119x; 1.3950x over previous
import jax
import jax.numpy as jnp
from jax.experimental import pallas as pl

_NEG = -100000000.0
_NHEADS = 2


def kernel(TX, X, MX, MY, params):
    f32 = jnp.float32
    Bn, Tn, Cn = X.shape
    L = params["chan_init"]["w"].shape[1]
    E = Tn * Cn
    dk = L // _NHEADS
    scale = 1.0 / (dk ** 0.5)

    mask = (MX + MY).astype(f32)

    pack3 = jnp.stack([X, MY, mask], axis=-1).reshape(Bn, E, 3)
    txc = TX[:, :, None]
    tnzc = (jnp.sum(mask, axis=2, keepdims=True) > 0).astype(f32)
    cnzc = (jnp.sum(mask, axis=1)[:, :, None] > 0).astype(f32)

    tw = params["time_init"]["w"]
    tb = params["time_init"]["b"][None, :]
    cw = params["chan_init"]["w"]
    cb = params["chan_init"]["b"][None, :]
    ew = jnp.concatenate([params["edge_init"]["w"],
                          jnp.zeros((1, L), f32)], axis=0)
    eb = params["edge_init"]["b"][None, :]

    def fuse_layer(lp):
        kw, kb = lp["attn"]["k"]["w"], lp["attn"]["k"]["b"]
        vw, vb = lp["attn"]["v"]["w"], lp["attn"]["v"]["b"]
        enw, enb = lp["edge_nn"]["w"], lp["edge_nn"]["b"]
        qw, qb = lp["attn"]["q"]["w"], lp["attn"]["q"]["b"]
        ow, ob = lp["attn"]["o"]["w"], lp["attn"]["o"]["b"][None, :]
        z = jnp.zeros((L, L), f32)
        w0 = jnp.concatenate(
            [z, kw[:L], z, vw[:L], enw[L:2 * L], qw, z], axis=1)
        w1 = jnp.concatenate(
            [kw[:L], z, vw[:L], z, enw[2 * L:3 * L], z, qw], axis=1)
        w2 = jnp.concatenate(
            [kw[L:], kw[L:], vw[L:], vw[L:], enw[:L], z, z], axis=1)
        bc = jnp.concatenate([kb, kb, vb, vb, enb, qb, qb])[None, :]
        return w0, w1, w2, bc, ow, ob

    layer_ws = [w for lp in params["layers"] for w in fuse_layer(lp)]
    n_layers = len(params["layers"])
    H = _NHEADS

    def body(pack_ref, txc_ref, tnzc_ref, cnzc_ref,
             tw_ref, tb_ref, cw_ref, cb_ref, ew_ref, eb_ref,
             *rest):
        lw_refs = rest[:-1]
        out_ref = rest[-1]

        pk = pack_ref[0]
        mc = pk[:, 2:3]
        mask3 = mc.reshape(Tn, Cn, 1)
        txk = txc_ref[0]
        tnz = tnzc_ref[0]
        cnz = cnzc_ref[0]

        d_i = jax.lax.broadcasted_iota(jnp.int32, (2 * L, 2 * H), 0)
        h_i = jax.lax.broadcasted_iota(jnp.int32, (2 * L, 2 * H), 1)
        S4 = (d_i // dk == h_i).astype(f32)
        d_j = jax.lax.broadcasted_iota(jnp.int32, (2 * H, 2 * L), 1)
        h_j = jax.lax.broadcasted_iota(jnp.int32, (2 * H, 2 * L), 0)
        ST4 = (d_j // dk == h_j).astype(f32)

        T_f = jnp.sin(txk * tw_ref[...] + tb_ref[...])
        C_f = jnp.maximum(cw_ref[...] + cb_ref[...], 0.0)
        U = jnp.maximum(jnp.dot(pk, ew_ref[...], preferred_element_type=f32)
                        + eb_ref[...], 0.0) * mc

        for li in range(n_layers):
            w0, w1, w2, bc, ow, ob = (
                r[...] for r in lw_refs[6 * li:6 * li + 6])

            TP = jnp.dot(T_f, w0, preferred_element_type=f32)
            CP = jnp.dot(C_f, w1, preferred_element_type=f32) + bc
            UP = jnp.dot(U, w2, preferred_element_type=f32)
            P3 = (UP.reshape(Tn, Cn, 7 * L)
                  + TP[:, None, :] + CP[None, :, :])

            KV = jnp.maximum(P3[:, :, 0:4 * L], 0.0)
            epre = P3[:, :, 4 * L:5 * L].reshape(E, L)
            Q2 = jnp.maximum(P3[:, :, 5 * L:7 * L], 0.0)

            prod = (Q2 * KV[:, :, 0:2 * L]).reshape(E, 2 * L)
            s4 = (jnp.dot(prod, S4, preferred_element_type=f32)
                  .reshape(Tn, Cn, 2 * H)) * scale
            s4 = jnp.where(mask3 > 0.0, s4, _NEG)

            sT = s4[:, :, 0:H]
            eT = jnp.exp(sT - jnp.max(sT, axis=1, keepdims=True))
            awT = eT / jnp.sum(eT, axis=1, keepdims=True)
            sC = s4[:, :, H:2 * H]
            eC = jnp.exp(sC - jnp.max(sC, axis=0, keepdims=True))
            awC = eC / jnp.sum(eC, axis=0, keepdims=True)

            aw4 = jnp.concatenate([awT, awC], axis=2).reshape(E, 2 * H)
            awb = jnp.dot(aw4, ST4, preferred_element_type=f32)
            WV = (awb.reshape(Tn, Cn, 2 * L)) * KV[:, :, 2 * L:4 * L]
            avT = jnp.sum(WV[:, :, 0:L], axis=1)
            avC = jnp.sum(WV[:, :, L:2 * L], axis=0)

            T_new = (jnp.dot(avT, ow, preferred_element_type=f32) + ob) * tnz
            C_new = (jnp.dot(avC, ow, preferred_element_type=f32) + ob) * cnz

            U = jnp.maximum(U + epre, 0.0) * mc
            T_f = T_new
            C_f = C_new

        out_ref[0] = U

    data_specs = [
        pl.BlockSpec((1, E, 3), lambda b: (b, 0, 0)),
        pl.BlockSpec((1, Tn, 1), lambda b: (b, 0, 0)),
        pl.BlockSpec((1, Tn, 1), lambda b: (b, 0, 0)),
        pl.BlockSpec((1, Cn, 1), lambda b: (b, 0, 0)),
    ]
    w_arrays = [tw, tb, cw, cb, ew, eb] + layer_ws
    w_specs = [pl.BlockSpec(a.shape, lambda b: (0, 0)) for a in w_arrays]

    out = pl.pallas_call(
        body,
        grid=(Bn,),
        in_specs=data_specs + w_specs,
        out_specs=pl.BlockSpec((1, E, L), lambda b: (b, 0, 0)),
        out_shape=jax.ShapeDtypeStruct((Bn, E, L), f32),
    )(pack3, txc, tnzc, cnzc, *w_arrays)
    return out.reshape(Bn, Tn, Cn, L)


# speedup vs baseline: 29.2713x; 1.1429x over previous
import jax
import jax.numpy as jnp
from jax.experimental import pallas as pl

_NEG = -100000000.0
_NHEADS = 2


def kernel(TX, X, MX, MY, params):
    f32 = jnp.float32
    Bn, Tn, Cn = X.shape
    L = params["chan_init"]["w"].shape[1]
    E = Tn * Cn
    dk = L // _NHEADS
    scale = 1.0 / (dk ** 0.5)
    H = _NHEADS
    n_layers = len(params["layers"])

    txc = TX[:, :, None]

    tw = params["time_init"]["w"]
    tb = params["time_init"]["b"][None, :]
    cw = params["chan_init"]["w"]
    cb = params["chan_init"]["b"][None, :]
    ew0 = params["edge_init"]["w"][0:1]
    ew1 = params["edge_init"]["w"][1:2]
    eb = params["edge_init"]["b"][None, :]

    layer_ws = []
    for lp in params["layers"]:
        layer_ws += [lp["attn"]["k"]["w"], lp["attn"]["k"]["b"][None, :],
                     lp["attn"]["v"]["w"], lp["attn"]["v"]["b"][None, :],
                     lp["edge_nn"]["w"], lp["edge_nn"]["b"][None, :],
                     lp["attn"]["q"]["w"], lp["attn"]["q"]["b"][None, :],
                     lp["attn"]["o"]["w"], lp["attn"]["o"]["b"][None, :]]

    def body(txc_ref, x_ref, mx_ref, my_ref,
             tw_ref, tb_ref, cw_ref, cb_ref, ew0_ref, ew1_ref, eb_ref,
             *rest):
        lw_refs = rest[:-1]
        out_ref = rest[-1]

        x2 = x_ref[0]
        my2 = my_ref[0]
        mask2 = mx_ref[0] + my2
        txk = txc_ref[0]

        maskb = jnp.broadcast_to(mask2[:, :, None], (Tn, Cn, L))
        mask3 = mask2[:, :, None]
        tnz = (jnp.sum(mask2, axis=1, keepdims=True) > 0).astype(f32)
        cnz = (jnp.sum(mask2.T, axis=1, keepdims=True) > 0).astype(f32)

        d_i = jax.lax.broadcasted_iota(jnp.int32, (2 * L, 2 * H), 0)
        h_i = jax.lax.broadcasted_iota(jnp.int32, (2 * L, 2 * H), 1)
        S4 = (d_i // dk == h_i).astype(f32)
        d_j = jax.lax.broadcasted_iota(jnp.int32, (2 * H, 2 * L), 1)
        h_j = jax.lax.broadcasted_iota(jnp.int32, (2 * H, 2 * L), 0)
        ST4 = (d_j // dk == h_j).astype(f32)

        T_f = jnp.sin(txk * tw_ref[...] + tb_ref[...])
        C_f = jnp.maximum(cw_ref[...] + cb_ref[...], 0.0)
        U = (jnp.maximum(x2[:, :, None] * ew0_ref[...][None]
                         + my2[:, :, None] * ew1_ref[...][None]
                         + eb_ref[...][None], 0.0)
             * maskb).reshape(E, L)
        maskE = maskb.reshape(E, L)

        for li in range(n_layers):
            kw, kb, vw, vb, enw, enb, qw, qb, ow, ob = (
                r[...] for r in lw_refs[10 * li:10 * li + 10])
            z = jnp.zeros((L, L), f32)
            w0 = jnp.concatenate(
                [z, kw[:L], z, vw[:L], enw[L:2 * L], qw, z], axis=1)
            w1 = jnp.concatenate(
                [kw[:L], z, vw[:L], z, enw[2 * L:3 * L], z, qw], axis=1)
            w2 = jnp.concatenate(
                [kw[L:], kw[L:], vw[L:], vw[L:], enw[:L]], axis=1)
            bc = jnp.concatenate([kb, kb, vb, vb, enb, qb, qb], axis=1)

            TP = jnp.dot(T_f, w0, preferred_element_type=f32)
            CP = jnp.dot(C_f, w1, preferred_element_type=f32) + bc
            UP = jnp.dot(U, w2, preferred_element_type=f32)
            TC3 = TP[:, None, :] + CP[None, :, :]
            P3 = (UP.reshape(Tn, Cn, 5 * L) + TC3[:, :, 0:5 * L])

            KV = jnp.maximum(P3[:, :, 0:4 * L], 0.0)
            epre = P3[:, :, 4 * L:5 * L].reshape(E, L)
            Q2 = jnp.maximum(TC3[:, :, 5 * L:7 * L], 0.0)

            prod = (Q2 * KV[:, :, 0:2 * L]).reshape(E, 2 * L)
            s4 = (jnp.dot(prod, S4, preferred_element_type=f32)
                  .reshape(Tn, Cn, 2 * H)) * scale
            s4 = jnp.where(mask3 > 0.0, s4, _NEG)

            sT = s4[:, :, 0:H]
            eT = jnp.exp(sT - jnp.max(sT, axis=1, keepdims=True))
            awT = eT / jnp.sum(eT, axis=1, keepdims=True)
            sC = s4[:, :, H:2 * H]
            eC = jnp.exp(sC - jnp.max(sC, axis=0, keepdims=True))
            awC = eC / jnp.sum(eC, axis=0, keepdims=True)

            aw4 = jnp.concatenate([awT, awC], axis=2).reshape(E, 2 * H)
            awb = jnp.dot(aw4, ST4, preferred_element_type=f32)
            WV = (awb.reshape(Tn, Cn, 2 * L)) * KV[:, :, 2 * L:4 * L]
            avT = jnp.sum(WV[:, :, 0:L], axis=1)
            avC = jnp.sum(WV[:, :, L:2 * L], axis=0)

            T_new = (jnp.dot(avT, ow, preferred_element_type=f32) + ob) * tnz
            C_new = (jnp.dot(avC, ow, preferred_element_type=f32) + ob) * cnz

            U = jnp.maximum(U + epre, 0.0) * maskE
            T_f = T_new
            C_f = C_new

        out_ref[0] = U

    data_specs = [
        pl.BlockSpec((1, Tn, 1), lambda b: (b, 0, 0)),
        pl.BlockSpec((1, Tn, Cn), lambda b: (b, 0, 0)),
        pl.BlockSpec((1, Tn, Cn), lambda b: (b, 0, 0)),
        pl.BlockSpec((1, Tn, Cn), lambda b: (b, 0, 0)),
    ]
    w_arrays = [tw, tb, cw, cb, ew0, ew1, eb] + layer_ws
    w_specs = [pl.BlockSpec(a.shape, lambda b: (0, 0)) for a in w_arrays]

    out = pl.pallas_call(
        body,
        grid=(Bn,),
        in_specs=data_specs + w_specs,
        out_specs=pl.BlockSpec((1, E, L), lambda b: (b, 0, 0)),
        out_shape=jax.ShapeDtypeStruct((Bn, E, L), f32),
    )(txc, X, MX, MY, *w_arrays)
    return out.reshape(Bn, Tn, Cn, L)
